# baseline (speedup 1.0000x reference)
import jax
import jax.numpy as jnp
from jax import lax
from jax.experimental import pallas as pl
from jax.experimental.pallas import tpu as pltpu

_CHUNK = 8


def _make_body(n_k, tile, a_dim, l_dim, l_pad, k_total):
    n_out = l_dim - 1

    def body(x1_ref, x2_ref, w_ref, bt_ref, scores_ref, *rest):
        out_refs = rest[:n_out]
        stt_ref, clse_ref, diff_ref = rest[n_out:n_out + 3]
        rem_refs = rest[n_out + 3:n_out + 10]
        sem_main, sem_rem = rest[n_out + 10:]
        i = pl.program_id(0)

        def main_copy(l):
            f8 = _CHUNK * (l // _CHUNK)
            return pltpu.make_async_copy(
                diff_ref.at[(l - 2) % 2, pl.ds(0, f8)],
                out_refs[l - 2].at[pl.ds(0, f8)],
                sem_main.at[(l - 2) % 2],
            )

        def rem_copy(l):
            f8, r = _CHUNK * (l // _CHUNK), l % _CHUNK
            return pltpu.make_async_copy(
                rem_refs[r - 1],
                out_refs[l - 2].at[pl.ds(f8, r)],
                sem_rem.at[r - 1],
            )

        @pl.when(i < n_k)
        def _phase1():
            x1 = x1_ref[...]
            x2 = x2_ref[...]
            w1 = w_ref[:a_dim, :]
            w2 = w_ref[a_dim:, :]
            st = lax.dot_general(w1, x1, (((0,), (1,)), ((), ())),
                                 preferred_element_type=jnp.float32)
            st = st + lax.dot_general(w2, x2, (((0,), (0,)), ((), ())),
                                      preferred_element_type=jnp.float32)
            st = st + bt_ref[...]
            scores_ref[...] = jnp.transpose(st)

            m_row = jnp.max(st, axis=0, keepdims=True)
            cs = jnp.exp(st - m_row)
            shift = 1
            while shift < l_dim:
                shifted = jnp.concatenate(
                    [jnp.zeros((shift, tile), jnp.float32), cs[:-shift, :]],
                    axis=0,
                )
                cs = cs + shifted
                shift *= 2
            clse = m_row + jnp.log(jnp.maximum(cs, jnp.float32(1e-37)))
            if l_pad > l_dim:
                pad = jnp.zeros((l_pad - l_dim, tile), jnp.float32)
                st = jnp.concatenate([st, pad], axis=0)
                clse = jnp.concatenate([clse, pad], axis=0)

            col = pl.multiple_of(i * tile, tile)
            stt_ref[:, pl.ds(col, tile)] = st
            clse_ref[:, pl.ds(col, tile)] = clse

        for ls in range(2, l_dim + 1):
            lp = ls - 2
            lr = ls - _CHUNK
            need_main = lp >= _CHUNK
            need_rem = lr >= 2 and lr % _CHUNK != 0
            if need_main or need_rem:
                @pl.when(i == n_k + ls - 2)
                def _wait(lp=lp, lr=lr, nm=need_main, nr=need_rem):
                    if nm:
                        main_copy(lp).wait()
                    if nr:
                        rem_copy(lr).wait()

        @pl.when(i >= n_k)
        def _fill():
            s2 = i - n_k
            l = s2 + 2
            buf = lax.rem(s2, 2)
            g8 = pl.multiple_of(((l - 1) // _CHUNK) * _CHUNK, _CHUNK)
            gran = clse_ref[pl.ds(g8, _CHUNK), :]
            ridx = jnp.broadcast_to(
                lax.rem(l - 1, _CHUNK), (1, k_total)
            ).astype(jnp.int32)
            clse_row = jnp.take_along_axis(gran, ridx, axis=0)
            nchunks = (l + _CHUNK - 1) // _CHUNK

            def fill(j, carry):
                rows = pl.ds(j * _CHUNK, _CHUNK)
                diff_ref[buf, rows, :] = stt_ref[rows, :] - clse_row
                return carry

            lax.fori_loop(0, nchunks, fill, 0)

        for ls in range(2, l_dim + 1):
            f8, r = _CHUNK * (ls // _CHUNK), ls % _CHUNK

            @pl.when(i == n_k + ls - 2)
            def _start(ls=ls, f8=f8, r=r):
                if r:
                    g = diff_ref[(ls - 2) % 2, f8:f8 + _CHUNK, :]
                    rem_refs[r - 1][...] = g[:r, :]
                    rem_copy(ls).start()
                if f8:
                    main_copy(ls).start()

        @pl.when(i == n_k + l_dim - 2)
        def _drain():
            for ls in (l_dim - 1, l_dim):
                if ls // _CHUNK:
                    main_copy(ls).wait()
            for ls in range(max(2, l_dim - _CHUNK + 1), l_dim + 1):
                if ls % _CHUNK:
                    rem_copy(ls).wait()

    return body


def _pick_tile(k_total):
    for t in (1024, 512, 256, 128, 64, 32, 16, 8):
        if k_total % t == 0:
            return t
    return k_total


def kernel(A_from, A_to, W, b):
    k_total, a_dim = A_from.shape
    l_dim = W.shape[1]
    tile = _pick_tile(k_total)
    n_k = k_total // tile
    l_pad = ((l_dim + _CHUNK - 1) // _CHUNK) * _CHUNK
    grid = (n_k + l_dim - 1,)

    bt = jnp.reshape(b.astype(jnp.float32), (l_dim, 1))

    out_shape = [jax.ShapeDtypeStruct((k_total, l_dim), jnp.float32)]
    out_specs = [pl.BlockSpec((tile, l_dim), lambda i: (jnp.minimum(i, n_k - 1), 0))]
    for l in range(2, l_dim + 1):
        out_shape.append(jax.ShapeDtypeStruct((l, k_total), jnp.float32))
        out_specs.append(pl.BlockSpec(memory_space=pl.ANY))

    scores, *lps = pl.pallas_call(
        _make_body(n_k, tile, a_dim, l_dim, l_pad, k_total),
        grid=grid,
        out_shape=tuple(out_shape),
        in_specs=[
            pl.BlockSpec((tile, a_dim), lambda i: (jnp.minimum(i, n_k - 1), 0)),
            pl.BlockSpec((a_dim, tile), lambda i: (0, jnp.minimum(i, n_k - 1))),
            pl.BlockSpec((2 * a_dim, l_dim), lambda i: (0, 0)),
            pl.BlockSpec((l_dim, 1), lambda i: (0, 0)),
        ],
        out_specs=tuple(out_specs),
        scratch_shapes=[
            pltpu.VMEM((l_pad, k_total), jnp.float32),
            pltpu.VMEM((l_pad, k_total), jnp.float32),
            pltpu.VMEM((2, l_pad, k_total), jnp.float32),
        ] + [
            pltpu.VMEM((r, k_total), jnp.float32) for r in range(1, _CHUNK)
        ] + [
            pltpu.SemaphoreType.DMA((2,)),
            pltpu.SemaphoreType.DMA((_CHUNK - 1,)),
        ],
        compiler_params=pltpu.CompilerParams(
            dimension_semantics=("arbitrary",),
            vmem_limit_bytes=56 * 1024 * 1024,
        ),
    )(A_from, A_to, W, bt)

    lplist = [jnp.zeros((1, k_total), jnp.float32)] + lps
    return lplist, scores


# speedup vs baseline: 1.3684x; 1.3684x over previous
import jax
import jax.numpy as jnp
from jax import lax
from jax.experimental import pallas as pl
from jax.experimental.pallas import tpu as pltpu


def _fused_kernel(x1_ref, x2_ref, w_ref, bt_ref, scores_ref, *out_refs):
    x1 = x1_ref[...]
    x2 = x2_ref[...]
    a_dim = x1.shape[1]
    w1 = w_ref[:a_dim, :]
    w2 = w_ref[a_dim:, :]

    st = lax.dot_general(w1, x1, (((0,), (1,)), ((), ())),
                         preferred_element_type=jnp.float32)
    st = st + lax.dot_general(w2, x2, (((0,), (0,)), ((), ())),
                              preferred_element_type=jnp.float32)
    st = st + bt_ref[...]

    scores_ref[...] = jnp.transpose(st)

    ll, tk = st.shape
    m_row = jnp.max(st, axis=0, keepdims=True)
    cs = jnp.exp(st - m_row)
    shift = 1
    while shift < ll:
        shifted = jnp.concatenate(
            [jnp.zeros((shift, tk), jnp.float32), cs[:-shift, :]], axis=0
        )
        cs = cs + shifted
        shift *= 2
    clse = m_row + jnp.log(jnp.maximum(cs, jnp.float32(1e-37)))

    for idx, l in enumerate(range(2, ll + 1)):
        out_refs[idx][...] = st[:l, :] - clse[l - 1:l, :]


def _pick_tile(k_total):
    for t in (2048, 1024, 512, 256, 128, 64, 32, 16, 8):
        if k_total % t == 0:
            return t
    return k_total


def kernel(A_from, A_to, W, b):
    k_total, a_dim = A_from.shape
    l_dim = W.shape[1]
    tile_k = _pick_tile(k_total)
    grid = (k_total // tile_k,)

    bt = jnp.reshape(b.astype(jnp.float32), (l_dim, 1))

    out_shape = [jax.ShapeDtypeStruct((k_total, l_dim), jnp.float32)]
    out_specs = [pl.BlockSpec((tile_k, l_dim), lambda i: (i, 0))]
    for l in range(2, l_dim + 1):
        out_shape.append(jax.ShapeDtypeStruct((l, k_total), jnp.float32))
        out_specs.append(pl.BlockSpec((l, tile_k), lambda i: (0, i)))

    scores, *lps = pl.pallas_call(
        _fused_kernel,
        grid=grid,
        out_shape=tuple(out_shape),
        in_specs=[
            pl.BlockSpec((tile_k, a_dim), lambda i: (i, 0)),
            pl.BlockSpec((a_dim, tile_k), lambda i: (0, i)),
            pl.BlockSpec((2 * a_dim, l_dim), lambda i: (0, 0)),
            pl.BlockSpec((l_dim, 1), lambda i: (0, 0)),
        ],
        out_specs=tuple(out_specs),
        compiler_params=pltpu.CompilerParams(
            dimension_semantics=("parallel",),
            vmem_limit_bytes=56 * 1024 * 1024,
        ),
    )(A_from, A_to, W, bt)

    lplist = [jnp.zeros((1, k_total), jnp.float32)] + lps
    return lplist, scores


# speedup vs baseline: 1.3827x; 1.0104x over previous
import jax
import jax.numpy as jnp
from jax import lax
from jax.experimental import pallas as pl
from jax.experimental.pallas import tpu as pltpu


def _fused_kernel(x1_ref, x2_ref, w_ref, b_ref, zero_ref, scores_ref, *out_refs):
    x1 = x1_ref[...]
    x2 = x2_ref[...]
    a_dim = x1.shape[1]
    w1 = w_ref[:a_dim, :]
    w2 = w_ref[a_dim:, :]

    st = lax.dot_general(w1, x1, (((0,), (1,)), ((), ())),
                         preferred_element_type=jnp.float32)
    st = st + lax.dot_general(w2, x2, (((0,), (0,)), ((), ())),
                              preferred_element_type=jnp.float32)
    st = st + jnp.transpose(b_ref[...])

    zero_ref[...] = jnp.zeros_like(zero_ref)
    scores_ref[...] = jnp.transpose(st)

    ll, tk = st.shape
    m_row = jnp.max(st, axis=0, keepdims=True)
    cs = jnp.exp(st - m_row)
    shift = 1
    while shift < ll:
        shifted = jnp.concatenate(
            [jnp.zeros((shift, tk), jnp.float32), cs[:-shift, :]], axis=0
        )
        cs = cs + shifted
        shift *= 2
    clse = m_row + jnp.log(jnp.maximum(cs, jnp.float32(1e-37)))

    for idx, l in enumerate(range(2, ll + 1)):
        out_refs[idx][...] = st[:l, :] - clse[l - 1:l, :]


def _pick_tile(k_total):
    for t in (2048, 1024, 512, 256, 128, 64, 32, 16, 8):
        if k_total % t == 0:
            return t
    return k_total


def kernel(A_from, A_to, W, b):
    k_total, a_dim = A_from.shape
    l_dim = W.shape[1]
    tile_k = _pick_tile(k_total)
    grid = (k_total // tile_k,)

    out_shape = [
        jax.ShapeDtypeStruct((1, k_total), jnp.float32),
        jax.ShapeDtypeStruct((k_total, l_dim), jnp.float32),
    ]
    out_specs = [
        pl.BlockSpec((1, tile_k), lambda i: (0, i)),
        pl.BlockSpec((tile_k, l_dim), lambda i: (i, 0)),
    ]
    for l in range(2, l_dim + 1):
        out_shape.append(jax.ShapeDtypeStruct((l, k_total), jnp.float32))
        out_specs.append(pl.BlockSpec((l, tile_k), lambda i: (0, i)))

    zrow, scores, *lps = pl.pallas_call(
        _fused_kernel,
        grid=grid,
        out_shape=tuple(out_shape),
        in_specs=[
            pl.BlockSpec((tile_k, a_dim), lambda i: (i, 0)),
            pl.BlockSpec((a_dim, tile_k), lambda i: (0, i)),
            pl.BlockSpec((2 * a_dim, l_dim), lambda i: (0, 0)),
            pl.BlockSpec((1, l_dim), lambda i: (0, 0)),
        ],
        out_specs=tuple(out_specs),
        compiler_params=pltpu.CompilerParams(
            dimension_semantics=("parallel",),
            vmem_limit_bytes=56 * 1024 * 1024,
        ),
    )(A_from, A_to, W, b.astype(jnp.float32))

    lplist = [zrow] + lps
    return lplist, scores
